# baseline (speedup 1.0000x reference)
import jax
import numpy as np
import jax.numpy as jnp
from jax import lax
from jax.experimental import pallas as pl
from jax.experimental.pallas import tpu as pltpu
from jax.experimental.pallas import tpu_sc as plsc

D_MODEL = 768
B = 4
S = 4096
EPS = 1e-12
NW = 32
P_PER_W = S // NW
G = 32
NCHUNK = (P_PER_W // G) * B
NJ = D_MODEL // 16


_GDN = lax.GatherDimensionNumbers(
    offset_dims=(), collapsed_slice_dims=(0,), start_index_map=(0,))


def _lane_perm(x, sh):
    perm = lax.iota(jnp.int32, 16) ^ sh
    return lax.gather(x, perm[:, None], _GDN, slice_sizes=(1,),
                      mode=lax.GatherScatterMode.PROMISE_IN_BOUNDS)


def _ln_tokens(rows_v, hbase, pos_v, pbase, g_v, b_v):
    @plsc.parallel_loop(0, G, 1, unroll=1)
    def token_body(t):
        r = hbase + t
        p = pbase + t
        acc = jnp.zeros((16,), jnp.float32)
        acc2 = jnp.zeros((16,), jnp.float32)
        for j in range(NJ):
            sl = pl.ds(j * 16, 16)
            x = rows_v[r, sl] + pos_v[p, sl]
            rows_v[r, sl] = x
            acc = acc + x
            acc2 = acc2 + x * x
        for sh in (8, 4, 2, 1):
            acc = acc + _lane_perm(acc, sh)
            acc2 = acc2 + _lane_perm(acc2, sh)
        mean_v = acc * (1.0 / D_MODEL)
        v = acc2 * (1.0 / D_MODEL) - mean_v * mean_v + EPS
        iv = plsc.bitcast(v, jnp.int32)
        iv = 0x5F3759DF - (iv >> 1)
        y = plsc.bitcast(iv, jnp.float32)
        half_v = 0.5 * v
        for _n in range(3):
            y = y * (1.5 - half_v * y * y)
        for j in range(NJ):
            sl = pl.ds(j * 16, 16)
            x = rows_v[r, sl]
            rows_v[r, sl] = (x - mean_v) * y


def _affine_tokens(rows_v, hbase, g_v, b_v):
    @plsc.parallel_loop(0, G, 1, unroll=1)
    def token_body(t):
        r = hbase + t
        for j in range(NJ):
            sl = pl.ds(j * 16, 16)
            rows_v[r, sl] = rows_v[r, sl] * g_v[sl] + b_v[sl]


def _sc_body(ids_hbm, wt_hbm, pt_hbm, g_hbm, b_hbm, out_hbm,
             idx_all, rows_v, pos_v, g_v, b_v, gsem, ssem):
    wid = lax.axis_index("s") * 2 + lax.axis_index("c")
    p0 = wid * P_PER_W
    pltpu.sync_copy(g_hbm, g_v)
    pltpu.sync_copy(b_hbm, b_v)
    dev = jnp.zeros((16,), jnp.float32)
    for j in range(NJ):
        sl = pl.ds(j * 16, 16)
        dev = dev + jnp.abs(g_v[sl] - 1.0) + jnp.abs(b_v[sl])
    nontrivial_gb = jnp.sum(dev) > 0.0
    for bb in range(B):
        pltpu.sync_copy(ids_hbm.at[pl.ds(bb * S + p0, P_PER_W)],
                        idx_all.at[pl.ds(bb * P_PER_W, P_PER_W)])

    def idx_off(c):
        return (c % B) * P_PER_W + (c // B) * G

    def tok_of(c):
        return (c % B) * S + p0 + (c // B) * G

    def start_gather(c):
        half = (c % 2) * G
        src = wt_hbm.at[idx_all.at[pl.ds(idx_off(c), G)]]
        pltpu.make_async_copy(src, rows_v.at[pl.ds(half, G)], gsem).start()

    def wait_rows(sem):
        pltpu.make_async_copy(wt_hbm.at[idx_all.at[pl.ds(0, G)]],
                              rows_v.at[pl.ds(0, G)], sem).wait()

    def start_store(c):
        half = (c % 2) * G
        pltpu.make_async_copy(rows_v.at[pl.ds(half, G)],
                              out_hbm.at[pl.ds(tok_of(c), G)], ssem).start()

    start_gather(0)

    def chunk_body(c, _):
        hbase = (c % 2) * G

        @pl.when(c + 1 < NCHUNK)
        def _():
            @pl.when(c >= 1)
            def _():
                wait_rows(ssem)

            start_gather(c + 1)

        wait_rows(gsem)

        @pl.when(lax.rem(c, B) == 0)
        def _():
            pltpu.sync_copy(pt_hbm.at[pl.ds(p0 + (c // B) * G, G)], pos_v)

        _ln_tokens(rows_v, hbase, pos_v, 0, g_v, b_v)

        @pl.when(nontrivial_gb)
        def _():
            _affine_tokens(rows_v, hbase, g_v, b_v)

        start_store(c)
        return 0

    lax.fori_loop(0, NCHUNK, chunk_body, 0)
    wait_rows(ssem)
    wait_rows(ssem)


@jax.jit
def _run(ids_flat, word_table, pos_table, gamma, beta):
    mesh = plsc.VectorSubcoreMesh(core_axis_name="c", subcore_axis_name="s", num_cores=2, num_subcores=16)
    k = pl.kernel(
        _sc_body,
        out_type=jax.ShapeDtypeStruct((B * S, D_MODEL), jnp.float32),
        mesh=mesh,
        compiler_params=pltpu.CompilerParams(needs_layout_passes=False),
        scratch_types=[
            pltpu.VMEM((B * P_PER_W,), jnp.int32),
            pltpu.VMEM((2 * G, D_MODEL), jnp.float32),
            pltpu.VMEM((G, D_MODEL), jnp.float32),
            pltpu.VMEM((D_MODEL,), jnp.float32),
            pltpu.VMEM((D_MODEL,), jnp.float32),
            pltpu.SemaphoreType.DMA,
            pltpu.SemaphoreType.DMA,
        ],
    )
    return k(ids_flat, word_table, pos_table, gamma, beta)


def kernel(input_ids, word_table, pos_table, gamma, beta):
    ids_flat = jnp.reshape(input_ids.astype(jnp.int32), (B * S,))
    out = _run(ids_flat, word_table, pos_table, gamma, beta)
    return jnp.reshape(out, (B, S, D_MODEL))


# speedup vs baseline: 1.0108x; 1.0108x over previous
import jax
import numpy as np
import jax.numpy as jnp
from jax import lax
from jax.experimental import pallas as pl
from jax.experimental.pallas import tpu as pltpu
from jax.experimental.pallas import tpu_sc as plsc

D_MODEL = 768
B = 4
S = 4096
EPS = 1e-12
NW = 32
P_PER_W = S // NW
G = 32
NCHUNK = (P_PER_W // G) * B
NJ = D_MODEL // 16


_GDN = lax.GatherDimensionNumbers(
    offset_dims=(), collapsed_slice_dims=(0,), start_index_map=(0,))


def _lane_perm(x, sh):
    perm = lax.iota(jnp.int32, 16) ^ sh
    return lax.gather(x, perm[:, None], _GDN, slice_sizes=(1,),
                      mode=lax.GatherScatterMode.PROMISE_IN_BOUNDS)


def _ln_tokens(rows_v, hbase, pos_v, pbase, g_v, b_v, apply_gb):
    @plsc.parallel_loop(0, G, 1, unroll=1)
    def token_body(t):
        r = hbase + t
        p = pbase + t
        acc = jnp.zeros((16,), jnp.float32)
        acc2 = jnp.zeros((16,), jnp.float32)
        for j in range(NJ):
            sl = pl.ds(j * 16, 16)
            x = rows_v[r, sl] + pos_v[p, sl]
            rows_v[r, sl] = x
            acc = acc + x
            acc2 = acc2 + x * x
        for sh in (8, 4, 2, 1):
            acc = acc + _lane_perm(acc, sh)
            acc2 = acc2 + _lane_perm(acc2, sh)
        mean_v = acc * (1.0 / D_MODEL)
        v = acc2 * (1.0 / D_MODEL) - mean_v * mean_v + EPS
        iv = plsc.bitcast(v, jnp.int32)
        iv = 0x5F3759DF - (iv >> 1)
        y = plsc.bitcast(iv, jnp.float32)
        half_v = 0.5 * v
        for _n in range(3):
            y = y * (1.5 - half_v * y * y)
        for j in range(NJ):
            sl = pl.ds(j * 16, 16)
            x = rows_v[r, sl]
            if apply_gb:
                rows_v[r, sl] = (x - mean_v) * y * g_v[sl] + b_v[sl]
            else:
                rows_v[r, sl] = (x - mean_v) * y


def _sc_body(ids_hbm, wt_hbm, pt_hbm, g_hbm, b_hbm, out_hbm,
             idx_all, rows_v, pos_v, g_v, b_v, gsem, ssem):
    wid = lax.axis_index("s") * 2 + lax.axis_index("c")
    p0 = wid * P_PER_W
    pltpu.sync_copy(g_hbm, g_v)
    pltpu.sync_copy(b_hbm, b_v)
    dev = jnp.zeros((16,), jnp.float32)
    for j in range(NJ):
        sl = pl.ds(j * 16, 16)
        dev = dev + jnp.abs(g_v[sl] - 1.0) + jnp.abs(b_v[sl])
    nontrivial_gb = jnp.sum(dev) > 0.0
    for bb in range(B):
        pltpu.sync_copy(ids_hbm.at[pl.ds(bb * S + p0, P_PER_W)],
                        idx_all.at[pl.ds(bb * P_PER_W, P_PER_W)])

    def idx_off(c):
        return (c % B) * P_PER_W + (c // B) * G

    def tok_of(c):
        return (c % B) * S + p0 + (c // B) * G

    def start_gather(c):
        half = (c % 2) * G
        src = wt_hbm.at[idx_all.at[pl.ds(idx_off(c), G)]]
        pltpu.make_async_copy(src, rows_v.at[pl.ds(half, G)], gsem).start()

    def wait_rows(sem):
        pltpu.make_async_copy(wt_hbm.at[idx_all.at[pl.ds(0, G)]],
                              rows_v.at[pl.ds(0, G)], sem).wait()

    def start_store(c):
        half = (c % 2) * G
        pltpu.make_async_copy(rows_v.at[pl.ds(half, G)],
                              out_hbm.at[pl.ds(tok_of(c), G)], ssem).start()

    start_gather(0)

    def run_chunks(apply_gb):
        def chunk_body(c, _):
            hbase = (c % 2) * G

            @pl.when(c + 1 < NCHUNK)
            def _():
                @pl.when(c >= 1)
                def _():
                    wait_rows(ssem)

                start_gather(c + 1)

            wait_rows(gsem)

            @pl.when(lax.rem(c, B) == 0)
            def _():
                pltpu.sync_copy(pt_hbm.at[pl.ds(p0 + (c // B) * G, G)], pos_v)

            _ln_tokens(rows_v, hbase, pos_v, 0, g_v, b_v, apply_gb)
            start_store(c)
            return 0

        lax.fori_loop(0, NCHUNK, chunk_body, 0)

    @pl.when(nontrivial_gb)
    def _():
        run_chunks(True)

    @pl.when(jnp.logical_not(nontrivial_gb))
    def _():
        run_chunks(False)

    wait_rows(ssem)
    wait_rows(ssem)


@jax.jit
def _run(ids_flat, word_table, pos_table, gamma, beta):
    mesh = plsc.VectorSubcoreMesh(core_axis_name="c", subcore_axis_name="s", num_cores=2, num_subcores=16)
    k = pl.kernel(
        _sc_body,
        out_type=jax.ShapeDtypeStruct((B * S, D_MODEL), jnp.float32),
        mesh=mesh,
        compiler_params=pltpu.CompilerParams(needs_layout_passes=False),
        scratch_types=[
            pltpu.VMEM((B * P_PER_W,), jnp.int32),
            pltpu.VMEM((2 * G, D_MODEL), jnp.float32),
            pltpu.VMEM((G, D_MODEL), jnp.float32),
            pltpu.VMEM((D_MODEL,), jnp.float32),
            pltpu.VMEM((D_MODEL,), jnp.float32),
            pltpu.SemaphoreType.DMA,
            pltpu.SemaphoreType.DMA,
        ],
    )
    return k(ids_flat, word_table, pos_table, gamma, beta)


def kernel(input_ids, word_table, pos_table, gamma, beta):
    ids_flat = jnp.reshape(input_ids.astype(jnp.int32), (B * S,))
    out = _run(ids_flat, word_table, pos_table, gamma, beta)
    return jnp.reshape(out, (B, S, D_MODEL))


# speedup vs baseline: 1.4002x; 1.3852x over previous
import jax
import numpy as np
import jax.numpy as jnp
from jax import lax
from jax.experimental import pallas as pl
from jax.experimental.pallas import tpu as pltpu
from jax.experimental.pallas import tpu_sc as plsc

D_MODEL = 768
B = 4
S = 4096
EPS = 1e-12
NW = 32
P_PER_W = S // NW
G = 32
NCHUNK = (P_PER_W // G) * B
NJ = D_MODEL // 16


_GDN = lax.GatherDimensionNumbers(
    offset_dims=(), collapsed_slice_dims=(0,), start_index_map=(0,))


def _lane_perm(x, sh):
    perm = lax.iota(jnp.int32, 16) ^ sh
    return lax.gather(x, perm[:, None], _GDN, slice_sizes=(1,),
                      mode=lax.GatherScatterMode.PROMISE_IN_BOUNDS)


def _ln_tokens(rows_v, hbase, pos_v, pbase, g_v, b_v):
    @plsc.parallel_loop(0, G, 1, unroll=1)
    def token_body(t):
        r = hbase + t
        p = pbase + t
        acc = jnp.zeros((16,), jnp.float32)
        acc2 = jnp.zeros((16,), jnp.float32)
        for j in range(NJ):
            sl = pl.ds(j * 16, 16)
            x = rows_v[r, sl] + pos_v[p, sl]
            rows_v[r, sl] = x
            acc = acc + x
            acc2 = acc2 + x * x
        for sh in (8, 4, 2, 1):
            acc = acc + _lane_perm(acc, sh)
            acc2 = acc2 + _lane_perm(acc2, sh)
        mean_v = acc * (1.0 / D_MODEL)
        v = acc2 * (1.0 / D_MODEL) - mean_v * mean_v + EPS
        iv = plsc.bitcast(v, jnp.int32)
        iv = 0x5F3759DF - (iv >> 1)
        y = plsc.bitcast(iv, jnp.float32)
        half_v = 0.5 * v
        for _n in range(3):
            y = y * (1.5 - half_v * y * y)
        for j in range(NJ):
            sl = pl.ds(j * 16, 16)
            x = rows_v[r, sl]
            rows_v[r, sl] = (x - mean_v) * y * g_v[sl] + b_v[sl]


def _sc_body(ids_hbm, wt_hbm, pt_hbm, g_hbm, b_hbm, out_hbm,
             idx_all, rows_v, pos_v, g_v, b_v, gsem, ssem):
    wid = lax.axis_index("s") * 2 + lax.axis_index("c")
    p0 = wid * P_PER_W
    pltpu.sync_copy(g_hbm, g_v)
    pltpu.sync_copy(b_hbm, b_v)
    for bb in range(B):
        pltpu.sync_copy(ids_hbm.at[pl.ds(bb * S + p0, P_PER_W)],
                        idx_all.at[pl.ds(bb * P_PER_W, P_PER_W)])

    def idx_off(c):
        return (c % B) * P_PER_W + (c // B) * G

    def tok_of(c):
        return (c % B) * S + p0 + (c // B) * G

    def start_gather(c):
        half = (c % 2) * G
        src = wt_hbm.at[idx_all.at[pl.ds(idx_off(c), G)]]
        pltpu.make_async_copy(src, rows_v.at[pl.ds(half, G)], gsem).start()

    def wait_rows(sem):
        pltpu.make_async_copy(wt_hbm.at[idx_all.at[pl.ds(0, G)]],
                              rows_v.at[pl.ds(0, G)], sem).wait()

    def start_store(c):
        half = (c % 2) * G
        pltpu.make_async_copy(rows_v.at[pl.ds(half, G)],
                              out_hbm.at[pl.ds(tok_of(c), G)], ssem).start()

    start_gather(0)

    def run_chunks():
        def chunk_body(c, _):
            hbase = (c % 2) * G

            @pl.when(c + 1 < NCHUNK)
            def _():
                @pl.when(c >= 1)
                def _():
                    wait_rows(ssem)

                start_gather(c + 1)

            wait_rows(gsem)

            @pl.when(lax.rem(c, B) == 0)
            def _():
                pltpu.sync_copy(pt_hbm.at[pl.ds(p0 + (c // B) * G, G)], pos_v)

            _ln_tokens(rows_v, hbase, pos_v, 0, g_v, b_v)
            start_store(c)
            return 0

        lax.fori_loop(0, NCHUNK, chunk_body, 0)

    run_chunks()
    wait_rows(ssem)
    wait_rows(ssem)


@jax.jit
def _run(ids_flat, word_table, pos_table, gamma, beta):
    mesh = plsc.VectorSubcoreMesh(core_axis_name="c", subcore_axis_name="s", num_cores=2, num_subcores=16)
    k = pl.kernel(
        _sc_body,
        out_type=jax.ShapeDtypeStruct((B * S, D_MODEL), jnp.float32),
        mesh=mesh,
        compiler_params=pltpu.CompilerParams(needs_layout_passes=False),
        scratch_types=[
            pltpu.VMEM((B * P_PER_W,), jnp.int32),
            pltpu.VMEM((2 * G, D_MODEL), jnp.float32),
            pltpu.VMEM((G, D_MODEL), jnp.float32),
            pltpu.VMEM((D_MODEL,), jnp.float32),
            pltpu.VMEM((D_MODEL,), jnp.float32),
            pltpu.SemaphoreType.DMA,
            pltpu.SemaphoreType.DMA,
        ],
    )
    return k(ids_flat, word_table, pos_table, gamma, beta)


def kernel(input_ids, word_table, pos_table, gamma, beta):
    ids_flat = jnp.reshape(input_ids.astype(jnp.int32), (B * S,))
    out = _run(ids_flat, word_table, pos_table, gamma, beta)
    return jnp.reshape(out, (B, S, D_MODEL))
